# Initial kernel scaffold; baseline (speedup 1.0000x reference)
import jax
import jax.numpy as jnp
from jax.experimental import pallas as pl


def kernel(x, orig_weight, new_embedding_weight):
    raise NotImplementedError("write your pallas kernel here")


# speedup vs baseline: 4.0973x; 4.0973x over previous
import functools

import jax
import jax.numpy as jnp
from jax import lax
from jax.experimental import pallas as pl
from jax.experimental.pallas import tpu as pltpu
from jax.experimental.pallas import tpu_sc as plsc

VOCAB = 100000
NEW = 200
D = 128
B = 4096
L = 200
TBL = VOCAB + NEW
N = B * L

NC = 2
NS = 16
NW = NC * NS
PER_W = N // NW
C = 128
NCH = PER_W // C


def _concat_body(orig_ref, new_ref, out_ref):
    i = pl.program_id(0)

    @pl.when(i < VOCAB // 200)
    def _():
        out_ref[...] = orig_ref[...]

    @pl.when(i == VOCAB // 200)
    def _():
        out_ref[...] = new_ref[1:, :]


_concat_table = pl.pallas_call(
    _concat_body,
    grid=(TBL // 200,),
    in_specs=[
        pl.BlockSpec((200, D), lambda i: (jnp.minimum(i, VOCAB // 200 - 1), 0)),
        pl.BlockSpec((NEW + 1, D), lambda i: (0, 0)),
    ],
    out_specs=pl.BlockSpec((200, D), lambda i: (i, 0)),
    out_shape=jax.ShapeDtypeStruct((TBL, D), jnp.float32),
)


_mesh = plsc.VectorSubcoreMesh(core_axis_name="c", subcore_axis_name="s")


@functools.partial(
    pl.kernel,
    mesh=_mesh,
    out_type=jax.ShapeDtypeStruct((N, D), jnp.float32),
    scratch_types=[
        pltpu.VMEM((NCH, C), jnp.int32),
        pltpu.VMEM((2, C, D), jnp.float32),
        pltpu.SemaphoreType.DMA,
        pltpu.SemaphoreType.DMA,
    ],
)
def _gather_kernel(table_hbm, idx_hbm, out_hbm, idx_v, rows_v, gsem, ssem):
    wid = lax.axis_index("s") * NC + lax.axis_index("c")
    base = wid * PER_W
    pltpu.sync_copy(idx_hbm.at[wid], idx_v)

    def body(g, carry):
        pltpu.async_copy(table_hbm.at[idx_v.at[g]], rows_v.at[0], gsem).wait()
        pltpu.sync_copy(rows_v.at[0], out_hbm.at[pl.ds(base + g * C, C)])
        return carry

    lax.fori_loop(0, NCH, body, 0)


def kernel(x, orig_weight, new_embedding_weight):
    table = _concat_table(orig_weight, new_embedding_weight)
    idx = x.astype(jnp.int32).reshape(NW, NCH, C)
    out = _gather_kernel(table, idx)
    return out.reshape(B, L, D)


# speedup vs baseline: 5.0897x; 1.2422x over previous
import functools

import jax
import jax.numpy as jnp
from jax import lax
from jax.experimental import pallas as pl
from jax.experimental.pallas import tpu as pltpu
from jax.experimental.pallas import tpu_sc as plsc

VOCAB = 100000
NEW = 200
D = 128
B = 4096
L = 200
TBL = VOCAB + NEW
N = B * L

NC = 2
NS = 16
NW = NC * NS
PER_W = N // NW
C = 128
NCH = PER_W // C
SUP = 2
NSUP = NCH // SUP


def _concat_body(orig_ref, new_ref, out_ref):
    i = pl.program_id(0)

    @pl.when(i < VOCAB // 200)
    def _():
        out_ref[...] = orig_ref[...]

    @pl.when(i == VOCAB // 200)
    def _():
        out_ref[...] = new_ref[1:, :]


_concat_table = pl.pallas_call(
    _concat_body,
    grid=(TBL // 200,),
    in_specs=[
        pl.BlockSpec((200, D), lambda i: (jnp.minimum(i, VOCAB // 200 - 1), 0)),
        pl.BlockSpec((NEW + 1, D), lambda i: (0, 0)),
    ],
    out_specs=pl.BlockSpec((200, D), lambda i: (i, 0)),
    out_shape=jax.ShapeDtypeStruct((TBL, D), jnp.float32),
)


_mesh = plsc.VectorSubcoreMesh(core_axis_name="c", subcore_axis_name="s")


@functools.partial(
    pl.kernel,
    mesh=_mesh,
    out_type=jax.ShapeDtypeStruct((N, D), jnp.float32),
    scratch_types=[
        pltpu.VMEM((NCH, C), jnp.int32),
        pltpu.VMEM((2, SUP * C, D), jnp.float32),
        pltpu.SemaphoreType.DMA,
        pltpu.SemaphoreType.DMA,
    ],
)
def _gather_kernel(table_hbm, idx_hbm, out_hbm, idx_v, rows_v, gsem, ssem):
    wid = lax.axis_index("s") * NC + lax.axis_index("c")
    base = wid * PER_W
    pltpu.sync_copy(idx_hbm.at[wid], idx_v)

    def fire_gather(s, b):
        for j in range(SUP):
            pltpu.async_copy(
                table_hbm.at[idx_v.at[s * SUP + j]],
                rows_v.at[b].at[pl.ds(j * C, C)],
                gsem,
            )

    def wait_gather(b):
        for j in range(SUP):
            pltpu.make_async_copy(
                table_hbm.at[idx_v.at[0]],
                rows_v.at[b].at[pl.ds(j * C, C)],
                gsem,
            ).wait()

    def fire_store(s, b):
        pltpu.async_copy(
            rows_v.at[b], out_hbm.at[pl.ds(base + s * SUP * C, SUP * C)], ssem
        )

    def wait_store(b):
        pltpu.make_async_copy(
            rows_v.at[b], out_hbm.at[pl.ds(base, SUP * C)], ssem
        ).wait()

    fire_gather(0, 0)

    def body(s, carry):
        b = s % 2
        wait_gather(b)

        @pl.when(s >= 1)
        def _():
            wait_store(1 - b)

        @pl.when(s + 1 < NSUP)
        def _():
            fire_gather(s + 1, 1 - b)

        fire_store(s, b)
        return carry

    lax.fori_loop(0, NSUP, body, 0)
    wait_store((NSUP - 1) % 2)


def kernel(x, orig_weight, new_embedding_weight):
    table = _concat_table(orig_weight, new_embedding_weight)
    idx = x.astype(jnp.int32).reshape(NW, NCH, C)
    out = _gather_kernel(table, idx)
    return out.reshape(B, L, D)


# speedup vs baseline: 5.7964x; 1.1388x over previous
import functools

import jax
import jax.numpy as jnp
from jax import lax
from jax.experimental import pallas as pl
from jax.experimental.pallas import tpu as pltpu
from jax.experimental.pallas import tpu_sc as plsc

VOCAB = 100000
NEW = 200
D = 128
B = 4096
L = 200
N = B * L

NC = 2
NS = 16
NW = NC * NS
PER_W = N // NW
C = 128
NCH = PER_W // C
SUP = 2
NSUP = NCH // SUP
LANES = 16

_mesh = plsc.VectorSubcoreMesh(core_axis_name="c", subcore_axis_name="s")


@functools.partial(
    pl.kernel,
    mesh=_mesh,
    out_type=jax.ShapeDtypeStruct((N, D), jnp.float32),
    scratch_types=[
        pltpu.VMEM((NCH, C), jnp.int32),
        pltpu.VMEM((2, SUP, C), jnp.int32),
        pltpu.VMEM((2, SUP * C, D), jnp.float32),
        pltpu.VMEM((NEW + 1, D), jnp.float32),
        pltpu.SemaphoreType.DMA,
        pltpu.SemaphoreType.DMA,
    ],
)
def _gather_kernel(orig_hbm, new_hbm, idx_hbm, out_hbm,
                   idx_v, clamp_v, rows_v, newtab_v, gsem, ssem):
    wid = lax.axis_index("s") * NC + lax.axis_index("c")
    base = wid * PER_W
    pltpu.sync_copy(idx_hbm.at[wid], idx_v)
    pltpu.sync_copy(new_hbm, newtab_v)

    def clamp(s, bb):
        for j in range(SUP):
            for q in range(C // LANES):
                v = idx_v[s * SUP + j, pl.ds(q * LANES, LANES)]
                clamp_v[bb, j, pl.ds(q * LANES, LANES)] = jnp.minimum(
                    v, VOCAB - 1
                )

    def fire_gather(s, bb):
        del s
        for j in range(SUP):
            pltpu.async_copy(
                orig_hbm.at[clamp_v.at[bb, j]],
                rows_v.at[bb].at[pl.ds(j * C, C)],
                gsem,
            )

    def wait_gather(bb):
        for j in range(SUP):
            pltpu.make_async_copy(
                orig_hbm.at[clamp_v.at[bb, j]],
                rows_v.at[bb].at[pl.ds(j * C, C)],
                gsem,
            ).wait()

    def fire_store(s, bb):
        pltpu.async_copy(
            rows_v.at[bb], out_hbm.at[pl.ds(base + s * SUP * C, SUP * C)], ssem
        )

    def wait_store(bb):
        pltpu.make_async_copy(
            rows_v.at[bb], out_hbm.at[pl.ds(base, SUP * C)], ssem
        ).wait()

    lane = lax.iota(jnp.int32, LANES)

    def vtake(x, i):
        return lax.gather(
            x,
            i[:, None],
            lax.GatherDimensionNumbers(
                offset_dims=(),
                collapsed_slice_dims=(0,),
                start_index_map=(0,),
            ),
            (1,),
            mode=lax.GatherScatterMode.PROMISE_IN_BOUNDS,
        )

    def fixup(s, bb):
        for j in range(SUP):
            for q in range(C // LANES):
                v = idx_v[s * SUP + j, pl.ds(q * LANES, LANES)]
                gv = v
                for sh in (8, 4, 2, 1):
                    gv = jnp.maximum(gv, vtake(gv, lane ^ sh))

                @pl.when(gv[0] >= VOCAB)
                def _(j=j, q=q):
                    def lane_body(jj, carry):
                        m = 1 - jnp.minimum(jnp.abs(lane - jj), 1)
                        sel = v * m - (1 - m)
                        for sh in (8, 4, 2, 1):
                            sel = jnp.maximum(sel, vtake(sel, lane ^ sh))
                        sj = sel[0]
                        mf = jnp.clip(sj - (VOCAB - 1), 0, 1).astype(
                            jnp.float32
                        )
                        r = jnp.clip(sj - VOCAB + 1, 0, NEW)
                        ro = j * C + q * LANES + jj
                        for h in range(D // LANES):
                            cur = rows_v[bb, ro, pl.ds(h * LANES, LANES)]
                            tb = newtab_v[r, pl.ds(h * LANES, LANES)]
                            rows_v[bb, ro, pl.ds(h * LANES, LANES)] = (
                                cur * (1.0 - mf) + tb * mf
                            )
                        return carry

                    lax.fori_loop(0, LANES, lane_body, 0)

    clamp(0, 0)
    fire_gather(0, 0)

    def body(s, carry):
        bb = s % 2

        @pl.when(s + 1 < NSUP)
        def _():
            clamp(s + 1, 1 - bb)

        wait_gather(bb)
        fixup(s, bb)

        @pl.when(s >= 1)
        def _():
            wait_store(1 - bb)

        @pl.when(s + 1 < NSUP)
        def _():
            fire_gather(s + 1, 1 - bb)

        fire_store(s, bb)
        return carry

    lax.fori_loop(0, NSUP, body, 0)
    wait_store((NSUP - 1) % 2)


def kernel(x, orig_weight, new_embedding_weight):
    idx = x.astype(jnp.int32).reshape(NW, NCH, C)
    out = _gather_kernel(orig_weight, new_embedding_weight, idx)
    return out.reshape(B, L, D)


# speedup vs baseline: 6.9781x; 1.2039x over previous
import functools

import jax
import jax.numpy as jnp
from jax import lax
from jax.experimental import pallas as pl
from jax.experimental.pallas import tpu as pltpu
from jax.experimental.pallas import tpu_sc as plsc

VOCAB = 100000
NEW = 200
D = 128
B = 4096
L = 200
N = B * L

NC = 2
NS = 16
NW = NC * NS
PER_W = N // NW
C = 128
NCH = PER_W // C
SUP = 2
NSUP = NCH // SUP
LANES = 16

_mesh = plsc.VectorSubcoreMesh(core_axis_name="c", subcore_axis_name="s")


@functools.partial(
    pl.kernel,
    mesh=_mesh,
    out_type=jax.ShapeDtypeStruct((N, D), jnp.float32),
    scratch_types=[
        pltpu.VMEM((NCH, C), jnp.int32),
        pltpu.VMEM((2, SUP, C), jnp.int32),
        pltpu.VMEM((2, SUP * C, D), jnp.float32),
        pltpu.VMEM((NEW + 1, D), jnp.float32),
        pltpu.SemaphoreType.DMA,
        pltpu.SemaphoreType.DMA,
    ],
)
def _gather_kernel(orig_hbm, new_hbm, idx_hbm, out_hbm,
                   idx_v, clamp_v, rows_v, newtab_v, gsem, ssem):
    wid = lax.axis_index("s") * NC + lax.axis_index("c")
    base = wid * PER_W
    pltpu.sync_copy(idx_hbm.at[wid], idx_v)
    pltpu.sync_copy(new_hbm, newtab_v)

    def clamp(s, bb):
        for j in range(SUP):
            for q in range(C // LANES):
                v = idx_v[s * SUP + j, pl.ds(q * LANES, LANES)]
                clamp_v[bb, j, pl.ds(q * LANES, LANES)] = jnp.minimum(
                    v, VOCAB - 1
                )

    def fire_gather(s, bb):
        del s
        for j in range(SUP):
            pltpu.async_copy(
                orig_hbm.at[clamp_v.at[bb, j]],
                rows_v.at[bb].at[pl.ds(j * C, C)],
                gsem,
            )

    def wait_gather(bb):
        for j in range(SUP):
            pltpu.make_async_copy(
                orig_hbm.at[clamp_v.at[bb, j]],
                rows_v.at[bb].at[pl.ds(j * C, C)],
                gsem,
            ).wait()

    def fire_store(s, bb):
        pltpu.async_copy(
            rows_v.at[bb], out_hbm.at[pl.ds(base + s * SUP * C, SUP * C)], ssem
        )

    def wait_store(bb):
        pltpu.make_async_copy(
            rows_v.at[bb], out_hbm.at[pl.ds(base, SUP * C)], ssem
        ).wait()

    lane = lax.iota(jnp.int32, LANES)

    def vtake(x, i):
        return lax.gather(
            x,
            i[:, None],
            lax.GatherDimensionNumbers(
                offset_dims=(),
                collapsed_slice_dims=(0,),
                start_index_map=(0,),
            ),
            (1,),
            mode=lax.GatherScatterMode.PROMISE_IN_BOUNDS,
        )

    def fixup(s, bb):
        for j in range(SUP):
            for q in range(C // LANES):
                v = idx_v[s * SUP + j, pl.ds(q * LANES, LANES)]
                gv = v
                for sh in (8, 4, 2, 1):
                    gv = jnp.maximum(gv, vtake(gv, lane ^ sh))

                @pl.when(gv[0] >= VOCAB)
                def _(j=j, q=q):
                    def lane_body(jj, carry):
                        m = 1 - jnp.minimum(jnp.abs(lane - jj), 1)
                        sel = v * m - (1 - m)
                        for sh in (8, 4, 2, 1):
                            sel = jnp.maximum(sel, vtake(sel, lane ^ sh))
                        sj = sel[0]
                        mf = jnp.clip(sj - (VOCAB - 1), 0, 1).astype(
                            jnp.float32
                        )
                        r = jnp.clip(sj - VOCAB + 1, 0, NEW)
                        ro = j * C + q * LANES + jj
                        for h in range(D // LANES):
                            cur = rows_v[bb, ro, pl.ds(h * LANES, LANES)]
                            tb = newtab_v[r, pl.ds(h * LANES, LANES)]
                            rows_v[bb, ro, pl.ds(h * LANES, LANES)] = (
                                cur * (1.0 - mf) + tb * mf
                            )
                        return carry

                    lax.fori_loop(0, LANES, lane_body, 0)

    clamp(0, 0)
    fire_gather(0, 0)

    def body(s, carry):
        bb = s % 2

        @pl.when(s + 1 < NSUP)
        def _():
            clamp(s + 1, 1 - bb)

        wait_gather(bb)

        @pl.when(s >= 1)
        def _():
            wait_store(1 - bb)

        @pl.when(s + 1 < NSUP)
        def _():
            fire_gather(s + 1, 1 - bb)

        fixup(s, bb)
        fire_store(s, bb)
        return carry

    lax.fori_loop(0, NSUP, body, 0)
    wait_store((NSUP - 1) % 2)


def kernel(x, orig_weight, new_embedding_weight):
    idx = x.astype(jnp.int32).reshape(NW, NCH, C)
    out = _gather_kernel(orig_weight, new_embedding_weight, idx)
    return out.reshape(B, L, D)
